# baseline (speedup 1.0000x reference)
import jax
import jax.numpy as jnp
from jax import lax
from jax.experimental import pallas as pl
from jax.experimental.pallas import tpu as pltpu
from jax.experimental.pallas import tpu_sc as plsc

_NE = 8
_K = 2
_D = 1024
_F = 2048
_BM = 512
_BF = 512
_RB = 512

_INTERPRET = False

_N = 4096
_S = _N * _K
_NW1 = 16
_NW2 = 32


def _vgather(v, idx):
    dnums = lax.GatherDimensionNumbers(
        offset_dims=(), collapsed_slice_dims=(0,), start_index_map=(0,))
    return lax.gather(v, idx[:, None], dnums, (1,),
                      mode=lax.GatherScatterMode.PROMISE_IN_BOUNDS)


def _cumsum16(v, lanes):
    for k in (1, 2, 4, 8):
        idx = jnp.maximum(lanes - k, 0)
        gate = jnp.clip(lanes - (k - 1), 0, 1)
        v = v + _vgather(v, idx) * gate
    return v


def _ind(v, e):
    d = v - e
    return 1 - jnp.minimum(d * d, 1)


def _hist_body(ef_hbm, hist_hbm, ef_v, bv):
    per = _S // _NW1
    wid = lax.axis_index("s")
    lanes = lax.broadcasted_iota(jnp.int32, (16,), 0)
    pltpu.sync_copy(ef_hbm.at[pl.ds(wid * per, per)], ef_v)
    last = lanes * 0 + 15
    hist = jnp.zeros((16,), jnp.int32)
    for c in range(per // 16):
        ev = ef_v[pl.ds(c * 16, 16)]
        for e in range(_NE):
            incl = _cumsum16(_ind(ev, e), lanes)
            pc = _vgather(incl, last)
            hist = hist + _ind(lanes, e) * pc
    bv[...] = hist
    pltpu.sync_copy(bv, hist_hbm.at[wid])


def _sc_hist(ef):
    mesh = plsc.VectorSubcoreMesh(
        core_axis_name="c", subcore_axis_name="s", num_cores=1)
    f = pl.kernel(
        _hist_body,
        out_type=jax.ShapeDtypeStruct((_NW1, 16), jnp.int32),
        mesh=mesh,
        scratch_types=[
            pltpu.VMEM((_S // _NW1,), jnp.int32),
            pltpu.VMEM((16,), jnp.int32),
        ],
        compiler_params=pltpu.CompilerParams(needs_layout_passes=False))
    return f(ef)


def _sort_body(ef_hbm, wf_hbm, hist_hbm, pe_hbm, po_hbm, cnt_hbm, sw_hbm,
               ef_v, wf_v, pos_v, pos2d, pev, pov, bv, pv, hv, sem):
    per = _S // _NW1
    ntok = per // _K
    wid = lax.axis_index("s")
    lanes = lax.broadcasted_iota(jnp.int32, (16,), 0)
    pltpu.sync_copy(ef_hbm.at[pl.ds(wid * per, per)], ef_v)
    pltpu.sync_copy(wf_hbm.at[pl.ds(wid * per, per)], wf_v)
    pltpu.sync_copy(hist_hbm, hv)
    last = lanes * 0 + 15

    total = jnp.zeros((16,), jnp.int32)
    pv[...] = jnp.zeros((16,), jnp.int32)
    for w in range(_NW1):
        r = hv[w]
        total = total + r

        @pl.when(w < wid)
        def _():
            pv[...] = pv[...] + r
    base = (_cumsum16(total, lanes) - total) + pv[...]

    @pl.when(wid == 0)
    def _():
        bv[...] = total
        pltpu.sync_copy(bv, cnt_hbm)

    for c in range(per // 16):
        ev = ef_v[pl.ds(c * 16, 16)]
        bv[...] = base
        bg = plsc.load_gather(bv, [ev])
        rank = jnp.zeros((16,), jnp.int32)
        add = jnp.zeros((16,), jnp.int32)
        for e in range(_NE):
            mi = _ind(ev, e)
            incl = _cumsum16(mi, lanes)
            rank = rank + mi * (incl - 1 - rank)
            pc = _vgather(incl, last)
            add = add + _ind(lanes, e) * pc
        pos = bg + rank
        pos_v[pl.ds(c * 16, 16)] = pos
        pos2d[c // 8, pl.ds((c % 8) * 16, 16)] = pos
        base = base + add

    cps = [pltpu.async_copy(wf_v.at[pl.ds(j * 128, 128)],
                            sw_hbm.at[pos2d.at[j]], sem)
           for j in range(4)]
    for cp in cps:
        cp.wait()

    for p in range(ntok // 16):
        idx = lanes * 2 + p * 32
        pev[p, :] = plsc.load_gather(pos_v, [idx])
        pov[p, :] = plsc.load_gather(pos_v, [idx + 1])
    nrow = ntok // 16
    pltpu.sync_copy(pev, pe_hbm.at[pl.ds(wid * nrow, nrow)])
    pltpu.sync_copy(pov, po_hbm.at[pl.ds(wid * nrow, nrow)])


def _sc_sort(ef, wf):
    hists = _sc_hist(ef)
    mesh = plsc.VectorSubcoreMesh(
        core_axis_name="c", subcore_axis_name="s", num_cores=1)
    per = _S // _NW1
    f = pl.kernel(
        _sort_body,
        out_type=[
            jax.ShapeDtypeStruct((_N // 16, 16), jnp.int32),
            jax.ShapeDtypeStruct((_N // 16, 16), jnp.int32),
            jax.ShapeDtypeStruct((16,), jnp.int32),
            jax.ShapeDtypeStruct((_S,), jnp.float32),
        ],
        mesh=mesh,
        scratch_types=[
            pltpu.VMEM((per,), jnp.int32),
            pltpu.VMEM((per,), jnp.float32),
            pltpu.VMEM((per,), jnp.int32),
            pltpu.VMEM((4, 128), jnp.int32),
            pltpu.VMEM((16, 16), jnp.int32),
            pltpu.VMEM((16, 16), jnp.int32),
            pltpu.VMEM((16,), jnp.int32),
            pltpu.VMEM((16,), jnp.int32),
            pltpu.VMEM((16, 16), jnp.int32),
            pltpu.SemaphoreType.DMA,
        ],
        compiler_params=pltpu.CompilerParams(needs_layout_passes=False))
    return f(ef, wf, hists)


def _scatter_rows_body(x_hbm, pe_hbm, po_hbm, out_hbm, pe_v, po_v, xb, s1, s2):
    tpw = _N // _NW2
    nrow = tpw // 16
    wid = lax.axis_index("s") * 2 + lax.axis_index("c")
    t0 = wid * tpw
    pltpu.sync_copy(pe_hbm.at[pl.ds(wid * nrow, nrow)], pe_v)
    pltpu.sync_copy(po_hbm.at[pl.ds(wid * nrow, nrow)], po_v)
    for mega in range(2):
        pltpu.sync_copy(x_hbm.at[pl.ds(t0 + mega * 64, 64)], xb)
        cps = []
        for j in range(4):
            r = mega * 4 + j
            src = xb.at[pl.ds(j * 16, 16)]
            cps.append(pltpu.async_copy(src, out_hbm.at[pe_v.at[r]], s1))
            cps.append(pltpu.async_copy(src, out_hbm.at[po_v.at[r]], s2))
        for cp in cps:
            cp.wait()


def _sc_scatter_rows(x_flat, pe2d, po2d):
    mesh = plsc.VectorSubcoreMesh(
        core_axis_name="c", subcore_axis_name="s", num_cores=2)
    f = pl.kernel(
        _scatter_rows_body,
        out_type=jax.ShapeDtypeStruct((_S, _D), jnp.float32),
        mesh=mesh,
        scratch_types=[
            pltpu.VMEM((_N // _NW2 // 16, 16), jnp.int32),
            pltpu.VMEM((_N // _NW2 // 16, 16), jnp.int32),
            pltpu.VMEM((64, _D), jnp.float32),
            pltpu.SemaphoreType.DMA,
            pltpu.SemaphoreType.DMA,
        ],
        compiler_params=pltpu.CompilerParams(needs_layout_passes=False))
    return f(x_flat, pe2d, po2d)


def _combine_body(so_hbm, pe_hbm, po_hbm, out_hbm, pe_v, po_v, be, bo, s1, s2):
    tpw = _N // _NW2
    nrow = tpw // 16
    wid = lax.axis_index("s") * 2 + lax.axis_index("c")
    t0 = wid * tpw
    pltpu.sync_copy(pe_hbm.at[pl.ds(wid * nrow, nrow)], pe_v)
    pltpu.sync_copy(po_hbm.at[pl.ds(wid * nrow, nrow)], po_v)
    for cch in range(nrow):
        pltpu.async_copy(so_hbm.at[pe_v.at[cch]], be, s1).wait()
        pltpu.async_copy(so_hbm.at[po_v.at[cch]], bo, s2).wait()

        def body(i, carry):
            for r in range(16):
                be[r, pl.ds(i * 16, 16)] = (
                    be[r, pl.ds(i * 16, 16)] + bo[r, pl.ds(i * 16, 16)])
            return carry

        lax.fori_loop(0, _D // 16, body, 0)
        pltpu.sync_copy(be, out_hbm.at[pl.ds(t0 + cch * 16, 16)])


def _sc_combine(sorted_out, pe2d, po2d):
    mesh = plsc.VectorSubcoreMesh(
        core_axis_name="c", subcore_axis_name="s", num_cores=2)
    f = pl.kernel(
        _combine_body,
        out_type=jax.ShapeDtypeStruct((_N, _D), jnp.float32),
        mesh=mesh,
        scratch_types=[
            pltpu.VMEM((_N // _NW2 // 16, 16), jnp.int32),
            pltpu.VMEM((_N // _NW2 // 16, 16), jnp.int32),
            pltpu.VMEM((16, _D), jnp.float32),
            pltpu.VMEM((16, _D), jnp.float32),
            pltpu.SemaphoreType.DMA,
            pltpu.SemaphoreType.DMA,
        ],
        compiler_params=pltpu.CompilerParams(needs_layout_passes=False))
    return f(sorted_out, pe2d, po2d)


def _router_body(x_ref, wr_ref, e_ref, w_ref):
    lt = jax.lax.dot_general(
        wr_ref[...], x_ref[...], (((1,), (1,)), ((), ())),
        preferred_element_type=jnp.float32)
    rows = jax.lax.broadcasted_iota(jnp.int32, lt.shape, 0)
    v1 = jnp.max(lt, axis=0)
    a1 = jnp.min(jnp.where(lt == v1[None, :], rows, _NE), axis=0)
    lt2 = jnp.where(rows == a1[None, :], -jnp.inf, lt)
    v2 = jnp.max(lt2, axis=0)
    a2 = jnp.min(jnp.where(lt2 == v2[None, :], rows, _NE), axis=0)
    p1 = 1.0 / (1.0 + jnp.exp(v2 - v1))
    e_ref[...] = jnp.concatenate([a1[None, :], a2[None, :]], axis=0)
    w_ref[...] = jnp.concatenate([p1[None, :], (1.0 - p1)[None, :]], axis=0)


def _route(x_flat, W_router):
    n = x_flat.shape[0]
    return pl.pallas_call(
        _router_body,
        grid=(n // _RB,),
        in_specs=[
            pl.BlockSpec((_RB, _D), lambda i: (i, 0)),
            pl.BlockSpec((_NE, _D), lambda i: (0, 0)),
        ],
        out_specs=[
            pl.BlockSpec((_K, _RB), lambda i: (0, i)),
            pl.BlockSpec((_K, _RB), lambda i: (0, i)),
        ],
        out_shape=[
            jax.ShapeDtypeStruct((_K, n), jnp.int32),
            jax.ShapeDtypeStruct((_K, n), jnp.float32),
        ],
        interpret=_INTERPRET,
    )(x_flat, W_router)


def _ffn_body(bid_ref, eid_ref, lo_ref, hi_ref,
              x_ref, w1_ref, w2_ref, w3_ref, sw_ref, o_ref, acc_ref):
    t = pl.program_id(0)
    f = pl.program_id(1)
    nf = pl.num_programs(1)
    lo = lo_ref[t]
    hi = hi_ref[t]

    @pl.when(hi > lo)
    def _():
        xb = x_ref[...].astype(jnp.bfloat16)
        g = jnp.dot(xb, w1_ref[0].astype(jnp.bfloat16),
                    preferred_element_type=jnp.float32)
        v = jnp.dot(xb, w2_ref[0].astype(jnp.bfloat16),
                    preferred_element_type=jnp.float32)
        h = (g * (1.0 / (1.0 + jnp.exp(-g)))) * v
        p = jnp.dot(h.astype(jnp.bfloat16), w3_ref[0].astype(jnp.bfloat16),
                    preferred_element_type=jnp.float32)

        @pl.when(f == 0)
        def _():
            acc_ref[...] = p

        @pl.when(f > 0)
        def _():
            acc_ref[...] = acc_ref[...] + p

        @pl.when(f == nf - 1)
        def _():
            r = jax.lax.broadcasted_iota(jnp.int32, (_BM, _D), 0)
            m = (r >= lo) & (r < hi)
            o_ref[...] = jnp.where(m, acc_ref[...] * sw_ref[...], o_ref[...])


def _grouped_ffn(sorted_inputs, w1, w2, w3, sorted_w, bid, eid, lo, hi):
    s = sorted_inputs.shape[0]
    nb = s // _BM
    t_tiles = nb + _NE
    nf = _F // _BF
    grid_spec = pltpu.PrefetchScalarGridSpec(
        num_scalar_prefetch=4,
        grid=(t_tiles, nf),
        in_specs=[
            pl.BlockSpec((_BM, _D), lambda t, f, b, e, l, h: (b[t], 0)),
            pl.BlockSpec((1, _D, _BF), lambda t, f, b, e, l, h: (e[t], 0, f)),
            pl.BlockSpec((1, _D, _BF), lambda t, f, b, e, l, h: (e[t], 0, f)),
            pl.BlockSpec((1, _BF, _D), lambda t, f, b, e, l, h: (e[t], f, 0)),
            pl.BlockSpec((_BM, 1), lambda t, f, b, e, l, h: (b[t], 0)),
        ],
        out_specs=pl.BlockSpec((_BM, _D), lambda t, f, b, e, l, h: (b[t], 0)),
        scratch_shapes=[pltpu.VMEM((_BM, _D), jnp.float32)],
    )
    return pl.pallas_call(
        _ffn_body,
        grid_spec=grid_spec,
        out_shape=jax.ShapeDtypeStruct((s, _D), jnp.float32),
        compiler_params=pltpu.CompilerParams(
            dimension_semantics=("arbitrary", "arbitrary")),
        interpret=_INTERPRET,
    )(bid, eid, lo, hi, sorted_inputs, w1, w2, w3,
      sorted_w.reshape(-1, 1))


def kernel(x, W_router, w1, w2, w3):
    batch, seq, d = x.shape
    x_flat = x.reshape(-1, d)
    n = x_flat.shape[0]
    s = n * _K

    e2, p2 = _route(x_flat, W_router)
    flat_e = e2.T.reshape(-1)
    flat_w = p2.T.reshape(-1)

    pe2d, po2d, counts16, sorted_w = _sc_sort(flat_e, flat_w)

    counts = counts16[:_NE]
    offsets = jnp.concatenate(
        [jnp.zeros((1,), jnp.int32), jnp.cumsum(counts, dtype=jnp.int32)])
    nb = s // _BM
    starts = jnp.sort(jnp.concatenate(
        [jnp.arange(nb, dtype=jnp.int32) * _BM, offsets[:_NE]]))
    ends = jnp.concatenate([starts[1:], jnp.array([s], jnp.int32)])
    bid = jnp.minimum(starts // _BM, nb - 1)
    eid = jnp.minimum(
        jnp.searchsorted(offsets[1:], starts, side='right'),
        _NE - 1).astype(jnp.int32)
    lo = starts - bid * _BM
    hi = ends - bid * _BM

    sorted_inputs = _sc_scatter_rows(x_flat, pe2d, po2d)
    sorted_out = _grouped_ffn(sorted_inputs, w1, w2, w3, sorted_w,
                              bid, eid, lo, hi)
    out_flat = _sc_combine(sorted_out, pe2d, po2d)
    return out_flat.reshape(batch, seq, d)


# speedup vs baseline: 1.2705x; 1.2705x over previous
import jax
import jax.numpy as jnp
from jax import lax
from jax.experimental import pallas as pl
from jax.experimental.pallas import tpu as pltpu
from jax.experimental.pallas import tpu_sc as plsc

_NE = 8
_K = 2
_D = 1024
_F = 2048
_BM = 256
_BF = 2048
_RB = 512

_INTERPRET = False

_N = 4096
_S = _N * _K
_NW1 = 32
_NW2 = 32


def _vgather(v, idx):
    dnums = lax.GatherDimensionNumbers(
        offset_dims=(), collapsed_slice_dims=(0,), start_index_map=(0,))
    return lax.gather(v, idx[:, None], dnums, (1,),
                      mode=lax.GatherScatterMode.PROMISE_IN_BOUNDS)


def _cumsum16(v, lanes):
    for k in (1, 2, 4, 8):
        idx = jnp.maximum(lanes - k, 0)
        gate = jnp.clip(lanes - (k - 1), 0, 1)
        v = v + _vgather(v, idx) * gate
    return v


def _ind(v, e):
    d = v - e
    return 1 - jnp.minimum(d * d, 1)


def _hist_body(ef_hbm, hist_hbm, ef_v, bv):
    per = _S // _NW1
    wid = lax.axis_index("s") * 2 + lax.axis_index("c")
    lanes = lax.broadcasted_iota(jnp.int32, (16,), 0)
    pltpu.sync_copy(ef_hbm.at[pl.ds(wid * per, per)], ef_v)
    last = lanes * 0 + 15
    hist = jnp.zeros((16,), jnp.int32)
    for c in range(per // 16):
        ev = ef_v[pl.ds(c * 16, 16)]
        for e in range(_NE):
            incl = _cumsum16(_ind(ev, e), lanes)
            pc = _vgather(incl, last)
            hist = hist + _ind(lanes, e) * pc
    bv[...] = hist
    pltpu.sync_copy(bv, hist_hbm.at[wid])


def _sc_hist(ef):
    mesh = plsc.VectorSubcoreMesh(
        core_axis_name="c", subcore_axis_name="s", num_cores=2)
    f = pl.kernel(
        _hist_body,
        out_type=jax.ShapeDtypeStruct((_NW1, 16), jnp.int32),
        mesh=mesh,
        scratch_types=[
            pltpu.VMEM((_S // _NW1,), jnp.int32),
            pltpu.VMEM((16,), jnp.int32),
        ],
        compiler_params=pltpu.CompilerParams(needs_layout_passes=False))
    return f(ef)


def _sort_body(ef_hbm, wf_hbm, hist_hbm, pe_hbm, po_hbm, cnt_hbm, sw_hbm,
               ef_v, wf_v, pos_v, pos2d, pev, pov, bv, pv, hv, sem):
    per = _S // _NW1
    ntok = per // _K
    wid = lax.axis_index("s") * 2 + lax.axis_index("c")
    lanes = lax.broadcasted_iota(jnp.int32, (16,), 0)
    pltpu.sync_copy(ef_hbm.at[pl.ds(wid * per, per)], ef_v)
    pltpu.sync_copy(wf_hbm.at[pl.ds(wid * per, per)], wf_v)
    pltpu.sync_copy(hist_hbm, hv)
    last = lanes * 0 + 15

    total = jnp.zeros((16,), jnp.int32)
    pv[...] = jnp.zeros((16,), jnp.int32)
    for w in range(_NW1):
        r = hv[w]
        total = total + r

        @pl.when(w < wid)
        def _():
            pv[...] = pv[...] + r
    base = (_cumsum16(total, lanes) - total) + pv[...]

    @pl.when(wid == 0)
    def _():
        bv[...] = total
        pltpu.sync_copy(bv, cnt_hbm)

    for c in range(per // 16):
        ev = ef_v[pl.ds(c * 16, 16)]
        bv[...] = base
        bg = plsc.load_gather(bv, [ev])
        rank = jnp.zeros((16,), jnp.int32)
        add = jnp.zeros((16,), jnp.int32)
        for e in range(_NE):
            mi = _ind(ev, e)
            incl = _cumsum16(mi, lanes)
            rank = rank + mi * (incl - 1 - rank)
            pc = _vgather(incl, last)
            add = add + _ind(lanes, e) * pc
        pos = bg + rank
        pos_v[pl.ds(c * 16, 16)] = pos
        pos2d[c // 8, pl.ds((c % 8) * 16, 16)] = pos
        base = base + add

    cps = [pltpu.async_copy(wf_v.at[pl.ds(j * 128, 128)],
                            sw_hbm.at[pos2d.at[j]], sem)
           for j in range(per // 128)]
    for cp in cps:
        cp.wait()

    for p in range(ntok // 16):
        idx = lanes * 2 + p * 32
        pev[p, :] = plsc.load_gather(pos_v, [idx])
        pov[p, :] = plsc.load_gather(pos_v, [idx + 1])
    nrow = ntok // 16
    pltpu.sync_copy(pev, pe_hbm.at[pl.ds(wid * nrow, nrow)])
    pltpu.sync_copy(pov, po_hbm.at[pl.ds(wid * nrow, nrow)])


def _sc_sort(ef, wf):
    hists = _sc_hist(ef)
    mesh = plsc.VectorSubcoreMesh(
        core_axis_name="c", subcore_axis_name="s", num_cores=2)
    per = _S // _NW1
    ntok = per // _K
    f = pl.kernel(
        _sort_body,
        out_type=[
            jax.ShapeDtypeStruct((_N // 16, 16), jnp.int32),
            jax.ShapeDtypeStruct((_N // 16, 16), jnp.int32),
            jax.ShapeDtypeStruct((16,), jnp.int32),
            jax.ShapeDtypeStruct((_S,), jnp.float32),
        ],
        mesh=mesh,
        scratch_types=[
            pltpu.VMEM((per,), jnp.int32),
            pltpu.VMEM((per,), jnp.float32),
            pltpu.VMEM((per,), jnp.int32),
            pltpu.VMEM((per // 128, 128), jnp.int32),
            pltpu.VMEM((ntok // 16, 16), jnp.int32),
            pltpu.VMEM((ntok // 16, 16), jnp.int32),
            pltpu.VMEM((16,), jnp.int32),
            pltpu.VMEM((16,), jnp.int32),
            pltpu.VMEM((_NW1, 16), jnp.int32),
            pltpu.SemaphoreType.DMA,
        ],
        compiler_params=pltpu.CompilerParams(needs_layout_passes=False))
    return f(ef, wf, hists)


def _scatter_rows_body(x_hbm, pe_hbm, po_hbm, out_hbm, pe_v, po_v, xb, s1, s2):
    tpw = _N // _NW2
    nrow = tpw // 16
    wid = lax.axis_index("s") * 2 + lax.axis_index("c")
    t0 = wid * tpw
    pltpu.sync_copy(pe_hbm.at[pl.ds(wid * nrow, nrow)], pe_v)
    pltpu.sync_copy(po_hbm.at[pl.ds(wid * nrow, nrow)], po_v)
    for mega in range(2):
        pltpu.sync_copy(x_hbm.at[pl.ds(t0 + mega * 64, 64)], xb)
        cps = []
        for j in range(4):
            r = mega * 4 + j
            src = xb.at[pl.ds(j * 16, 16)]
            cps.append(pltpu.async_copy(src, out_hbm.at[pe_v.at[r]], s1))
            cps.append(pltpu.async_copy(src, out_hbm.at[po_v.at[r]], s2))
        for cp in cps:
            cp.wait()


def _sc_scatter_rows(x_flat, pe2d, po2d):
    mesh = plsc.VectorSubcoreMesh(
        core_axis_name="c", subcore_axis_name="s", num_cores=2)
    f = pl.kernel(
        _scatter_rows_body,
        out_type=jax.ShapeDtypeStruct((_S, _D), jnp.float32),
        mesh=mesh,
        scratch_types=[
            pltpu.VMEM((_N // _NW2 // 16, 16), jnp.int32),
            pltpu.VMEM((_N // _NW2 // 16, 16), jnp.int32),
            pltpu.VMEM((64, _D), jnp.float32),
            pltpu.SemaphoreType.DMA,
            pltpu.SemaphoreType.DMA,
        ],
        compiler_params=pltpu.CompilerParams(needs_layout_passes=False))
    return f(x_flat, pe2d, po2d)


def _combine_body(so_hbm, pe_hbm, po_hbm, out_hbm, pe_v, po_v, be, bo, s1, s2):
    tpw = _N // _NW2
    nrow = tpw // 16
    wid = lax.axis_index("s") * 2 + lax.axis_index("c")
    t0 = wid * tpw
    pltpu.sync_copy(pe_hbm.at[pl.ds(wid * nrow, nrow)], pe_v)
    pltpu.sync_copy(po_hbm.at[pl.ds(wid * nrow, nrow)], po_v)
    for cch in range(nrow):
        cpe = pltpu.async_copy(so_hbm.at[pe_v.at[cch]], be, s1)
        cpo = pltpu.async_copy(so_hbm.at[po_v.at[cch]], bo, s2)
        cpe.wait()
        cpo.wait()

        def body(i, carry):
            for r in range(16):
                be[r, pl.ds(i * 16, 16)] = (
                    be[r, pl.ds(i * 16, 16)] + bo[r, pl.ds(i * 16, 16)])
            return carry

        lax.fori_loop(0, _D // 16, body, 0)
        pltpu.sync_copy(be, out_hbm.at[pl.ds(t0 + cch * 16, 16)])


def _sc_combine(sorted_out, pe2d, po2d):
    mesh = plsc.VectorSubcoreMesh(
        core_axis_name="c", subcore_axis_name="s", num_cores=2)
    f = pl.kernel(
        _combine_body,
        out_type=jax.ShapeDtypeStruct((_N, _D), jnp.float32),
        mesh=mesh,
        scratch_types=[
            pltpu.VMEM((_N // _NW2 // 16, 16), jnp.int32),
            pltpu.VMEM((_N // _NW2 // 16, 16), jnp.int32),
            pltpu.VMEM((16, _D), jnp.float32),
            pltpu.VMEM((16, _D), jnp.float32),
            pltpu.SemaphoreType.DMA,
            pltpu.SemaphoreType.DMA,
        ],
        compiler_params=pltpu.CompilerParams(needs_layout_passes=False))
    return f(sorted_out, pe2d, po2d)


def _router_body(x_ref, wr_ref, e_ref, w_ref):
    lt = jax.lax.dot_general(
        wr_ref[...], x_ref[...], (((1,), (1,)), ((), ())),
        preferred_element_type=jnp.float32)
    rows = jax.lax.broadcasted_iota(jnp.int32, lt.shape, 0)
    v1 = jnp.max(lt, axis=0)
    a1 = jnp.min(jnp.where(lt == v1[None, :], rows, _NE), axis=0)
    lt2 = jnp.where(rows == a1[None, :], -jnp.inf, lt)
    v2 = jnp.max(lt2, axis=0)
    a2 = jnp.min(jnp.where(lt2 == v2[None, :], rows, _NE), axis=0)
    p1 = 1.0 / (1.0 + jnp.exp(v2 - v1))
    e_ref[...] = jnp.concatenate([a1[None, :], a2[None, :]], axis=0)
    w_ref[...] = jnp.concatenate([p1[None, :], (1.0 - p1)[None, :]], axis=0)


def _route(x_flat, W_router):
    n = x_flat.shape[0]
    return pl.pallas_call(
        _router_body,
        grid=(n // _RB,),
        in_specs=[
            pl.BlockSpec((_RB, _D), lambda i: (i, 0)),
            pl.BlockSpec((_NE, _D), lambda i: (0, 0)),
        ],
        out_specs=[
            pl.BlockSpec((_K, _RB), lambda i: (0, i)),
            pl.BlockSpec((_K, _RB), lambda i: (0, i)),
        ],
        out_shape=[
            jax.ShapeDtypeStruct((_K, n), jnp.int32),
            jax.ShapeDtypeStruct((_K, n), jnp.float32),
        ],
        interpret=_INTERPRET,
    )(x_flat, W_router)


def _ffn_body(bid_ref, eid_ref, lo_ref, hi_ref,
              x_ref, w1_ref, w2_ref, w3_ref, sw_ref, o_ref, acc_ref):
    t = pl.program_id(0)
    f = pl.program_id(1)
    nf = pl.num_programs(1)
    lo = lo_ref[t]
    hi = hi_ref[t]

    @pl.when(hi > lo)
    def _():
        xb = x_ref[...]
        g = jnp.dot(xb, w1_ref[0], preferred_element_type=jnp.float32)
        v = jnp.dot(xb, w2_ref[0], preferred_element_type=jnp.float32)
        h = (g * (1.0 / (1.0 + jnp.exp(-g)))) * v
        p = jnp.dot(h, w3_ref[0], preferred_element_type=jnp.float32)

        @pl.when(f == 0)
        def _():
            acc_ref[...] = p

        @pl.when(f > 0)
        def _():
            acc_ref[...] = acc_ref[...] + p

        @pl.when(f == nf - 1)
        def _():
            r = jax.lax.broadcasted_iota(jnp.int32, (_BM, _D), 0)
            m = (r >= lo) & (r < hi)
            o_ref[...] = jnp.where(m, acc_ref[...] * sw_ref[...], o_ref[...])


def _grouped_ffn(sorted_inputs, w1, w2, w3, sorted_w, bid, eid, lo, hi):
    s = sorted_inputs.shape[0]
    nb = s // _BM
    t_tiles = nb + _NE
    nf = _F // _BF
    grid_spec = pltpu.PrefetchScalarGridSpec(
        num_scalar_prefetch=4,
        grid=(t_tiles, nf),
        in_specs=[
            pl.BlockSpec((_BM, _D), lambda t, f, b, e, l, h: (b[t], 0)),
            pl.BlockSpec((1, _D, _BF), lambda t, f, b, e, l, h: (e[t], 0, f)),
            pl.BlockSpec((1, _D, _BF), lambda t, f, b, e, l, h: (e[t], 0, f)),
            pl.BlockSpec((1, _BF, _D), lambda t, f, b, e, l, h: (e[t], f, 0)),
            pl.BlockSpec((_BM, 1), lambda t, f, b, e, l, h: (b[t], 0)),
        ],
        out_specs=pl.BlockSpec((_BM, _D), lambda t, f, b, e, l, h: (b[t], 0)),
        scratch_shapes=[pltpu.VMEM((_BM, _D), jnp.float32)],
    )
    return pl.pallas_call(
        _ffn_body,
        grid_spec=grid_spec,
        out_shape=jax.ShapeDtypeStruct((s, _D), jnp.float32),
        compiler_params=pltpu.CompilerParams(
            dimension_semantics=("arbitrary", "arbitrary")),
        interpret=_INTERPRET,
    )(bid, eid, lo, hi, sorted_inputs, w1, w2, w3,
      sorted_w.reshape(-1, 1))


def kernel(x, W_router, w1, w2, w3):
    batch, seq, d = x.shape
    x_flat = x.reshape(-1, d)
    n = x_flat.shape[0]
    s = n * _K

    e2, p2 = _route(x_flat, W_router)
    flat_e = e2.T.reshape(-1)
    flat_w = p2.T.reshape(-1)

    pe2d, po2d, counts16, sorted_w = _sc_sort(flat_e, flat_w)

    counts = counts16[:_NE]
    offsets = jnp.concatenate(
        [jnp.zeros((1,), jnp.int32), jnp.cumsum(counts, dtype=jnp.int32)])
    nb = s // _BM
    starts = jnp.sort(jnp.concatenate(
        [jnp.arange(nb, dtype=jnp.int32) * _BM, offsets[:_NE]]))
    ends = jnp.concatenate([starts[1:], jnp.array([s], jnp.int32)])
    bid = jnp.minimum(starts // _BM, nb - 1)
    eid = jnp.minimum(
        jnp.searchsorted(offsets[1:], starts, side='right'),
        _NE - 1).astype(jnp.int32)
    lo = starts - bid * _BM
    hi = ends - bid * _BM

    sorted_inputs = _sc_scatter_rows(x_flat, pe2d, po2d)
    sorted_out = _grouped_ffn(sorted_inputs, w1, w2, w3, sorted_w,
                              bid, eid, lo, hi)
    out_flat = _sc_combine(sorted_out, pe2d, po2d)
    return out_flat.reshape(batch, seq, d)
